# baseline (speedup 1.0000x reference)
import dataclasses
import functools

import jax
import jax.numpy as jnp
from jax import lax
from jax.experimental import pallas as pl
from jax.experimental.pallas import tpu as pltpu
from jax.experimental.pallas import tpu_sc as plsc

_B = 16384
_D = 256
_E = 256
_NC = 2
_NS = 16
_NW = _NC * _NS
_BPT = _B // _NW
_C = 8
_NSTEP = _BPT // _C
_SCH = 64
_NSCH = _BPT // _SCH
_GRP = 8
_NGRP = _NSTEP // _GRP

_BT = 2048
_NBT = _B // _BT

_sc_mesh = plsc.VectorSubcoreMesh(core_axis_name="c", subcore_axis_name="s")

_sc_params = pltpu.CompilerParams()
if "needs_layout_passes" in pltpu.CompilerParams.__dataclass_fields__:
    _sc_params = dataclasses.replace(_sc_params, needs_layout_passes=False)


@functools.partial(
    pl.kernel,
    mesh=_sc_mesh,
    compiler_params=_sc_params,
    out_type=(
        jax.ShapeDtypeStruct((_B, _D), jnp.float32),
        jax.ShapeDtypeStruct((_B, _D), jnp.float32),
    ),
    scratch_types=[
        pltpu.VMEM((2, _GRP * _C, 10), jnp.int32),
        pltpu.VMEM((_BPT,), jnp.int32),
        pltpu.VMEM((2, 10 * _C), jnp.int32),
        pltpu.VMEM((2, 10 * _C), jnp.int32),
        pltpu.VMEM((2, 10 * _C, _D), jnp.float32),
        pltpu.VMEM((2, _C, _D), jnp.float32),
        pltpu.VMEM((2, _SCH, _D), jnp.float32),
        pltpu.SemaphoreType.DMA,
        pltpu.SemaphoreType.DMA,
        pltpu.SemaphoreType.DMA,
        pltpu.SemaphoreType.DMA,
        pltpu.SemaphoreType.DMA,
        pltpu.SemaphoreType.DMA,
        pltpu.SemaphoreType.DMA,
        pltpu.SemaphoreType.DMA,
    ],
)
def _sc_gather_sum(table_hbm, nidx_hbm, sidx_hbm, pat_hbm,
                   self_hbm, nsum_hbm,
                   gidx, sidx_v, pat_v, fidx, bufs, obuf, sbuf,
                   g0, g1, o0, o1, sg, sw, i0, i1):
    cid = lax.axis_index("c")
    sid = lax.axis_index("s")
    wid = sid * _NC + cid

    pltpu.sync_copy(sidx_hbm.at[pl.ds(wid * _BPT, _BPT)], sidx_v)
    pltpu.sync_copy(pat_hbm, pat_v)

    def idx_load(g, q, sem):
        return pltpu.make_async_copy(
            nidx_hbm.at[pl.ds(wid * _BPT + g * (_GRP * _C), _GRP * _C)],
            gidx.at[q], sem)

    def flatten_idx(c, p, q):
        rbase = (c % _GRP) * _C
        for k in range(10 * _C // 16):
            sl = pl.ds(k * 16, 16)
            rv = pat_v[0, sl] + rbase
            cv = pat_v[1, sl]
            fidx[p, sl] = plsc.load_gather(gidx.at[q], [rv, cv])

    def ngather(c, p, sem):
        return pltpu.make_async_copy(
            table_hbm.at[fidx.at[p]],
            bufs.at[p], sem)

    def sgather(c, p, sem):
        return pltpu.make_async_copy(
            table_hbm.at[sidx_v.at[pl.ds(c * _SCH, _SCH)]], sbuf.at[p], sem)

    def swrite(c, p, sem):
        return pltpu.make_async_copy(
            sbuf.at[p], self_hbm.at[pl.ds(wid * _BPT + c * _SCH, _SCH)], sem)

    def copyout(c, p, sem):
        return pltpu.make_async_copy(
            obuf.at[p],
            nsum_hbm.at[pl.ds(wid * _BPT + c * _C, _C)], sem)

    def sum_rows(p):
        buf = bufs.at[p]
        ob = obuf.at[p]

        def tree(x):
            t0 = x[0] + x[1]
            t1 = x[2] + x[3]
            t2 = x[4] + x[5]
            t3 = x[6] + x[7]
            t4 = x[8] + x[9]
            return ((t0 + t1) + (t2 + t3)) + t4

        @plsc.parallel_loop(0, _C, unroll=2)
        def _(b):
            r0 = b * 10
            for k in range(0, _D // 16, 2):
                sl0 = pl.ds(k * 16, 16)
                sl1 = pl.ds((k + 1) * 16, 16)
                xa = [buf[r0 + s, sl0] for s in range(10)]
                xb = [buf[r0 + s, sl1] for s in range(10)]
                va = tree(xa)
                vb = tree(xb)
                ob[b, sl0] = va
                ob[b, sl1] = vb

    idx_load(0, 0, i0).start()
    idx_load(0, 0, i0).wait()
    idx_load(1, 1, i1).start()
    flatten_idx(0, 0, 0)
    ngather(0, 0, g0).start()
    flatten_idx(1, 1, 0)
    ngather(1, 1, g1).start()

    @pl.loop(0, _NSTEP // 2)
    def _(i):
        for p, gs, os in ((0, g0, o0), (1, g1, o1)):
            c = i * 2 + p
            ngather(c, p, gs).wait()

            @pl.when(i > 0)
            def _():
                copyout(c - 2, p, os).wait()

            sum_rows(p)
            copyout(c, p, os).start()
            nc = c + 2

            @pl.when(nc < _NSTEP)
            def _():
                ng = nc // _GRP
                for qn, s_this, s_next in ((0, i0, i1), (1, i1, i0)):
                    @pl.when(jnp.logical_and(nc % _GRP == 0,
                                             ng % 2 == qn))
                    def _():
                        idx_load(ng, qn, s_this).wait()

                        @pl.when(ng + 1 < _NGRP)
                        def _():
                            idx_load(ng + 1, 1 - qn, s_next).start()

                for qf in (0, 1):
                    @pl.when(ng % 2 == qf)
                    def _():
                        flatten_idx(nc, p, qf)
                        ngather(nc, p, gs).start()

    copyout(_NSTEP - 2, 0, o0).wait()
    copyout(_NSTEP - 1, 1, o1).wait()

    sgather(0, 0, sg).start()
    sgather(1, 1, sw).start()

    @pl.loop(0, _NSCH // 2)
    def _(i):
        for p, gs, ws in ((0, sg, o0), (1, sw, o1)):
            c = i * 2 + p
            sgather(c, p, gs).wait()
            swrite(c, p, ws).start()
            swrite(c, p, ws).wait()
            nc = c + 2

            @pl.when(nc < _NSCH)
            def _():
                sgather(nc, p, gs).start()


def _tc_body(s_ref, n_ref, w_ref, o_ref):
    self_f = s_ref[...]
    neigh = n_ref[...] * jnp.float32(0.1)
    w = w_ref[...]
    acc = lax.dot_general(
        w[:, :_D], self_f, (((1,), (1,)), ((), ())),
        preferred_element_type=jnp.float32, precision=lax.Precision.DEFAULT)
    acc = acc + lax.dot_general(
        w[:, _D:], neigh, (((1,), (1,)), ((), ())),
        preferred_element_type=jnp.float32, precision=lax.Precision.DEFAULT)
    o_ref[...] = jnp.maximum(acc, jnp.float32(0.0))


def _tc_matmul(self_rows, nsum_rows, weight):
    return pl.pallas_call(
        _tc_body,
        grid=(_NBT,),
        in_specs=[
            pl.BlockSpec((_BT, _D), lambda i: (i, 0)),
            pl.BlockSpec((_BT, _D), lambda i: (i, 0)),
            pl.BlockSpec((_E, 2 * _D), lambda i: (0, 0)),
        ],
        out_specs=pl.BlockSpec((_E, _BT), lambda i: (0, i)),
        out_shape=jax.ShapeDtypeStruct((_E, _B), jnp.float32),
    )(self_rows, nsum_rows, weight)


def kernel(features, nodes, neigh_idx, weight):
    ar = jnp.arange(10 * _C, dtype=jnp.int32)
    pat = jnp.stack([ar // 10, ar % 10])
    self_rows, nsum_rows = _sc_gather_sum(features, neigh_idx, nodes, pat)
    return _tc_matmul(self_rows, nsum_rows, weight)


# speedup vs baseline: 1.2498x; 1.2498x over previous
import dataclasses
import functools

import jax
import jax.numpy as jnp
from jax import lax
from jax.experimental import pallas as pl
from jax.experimental.pallas import tpu as pltpu
from jax.experimental.pallas import tpu_sc as plsc

_B = 16384
_D = 256
_E = 256
_NC = 2
_NS = 16
_NW = _NC * _NS
_BPT = _B // _NW
_C = 8
_NSTEP = _BPT // _C
_SCH = 64
_NSCH = _BPT // _SCH
_GRP = 8
_NGRP = _NSTEP // _GRP

_BT = 2048
_NBT = _B // _BT

_sc_mesh = plsc.VectorSubcoreMesh(core_axis_name="c", subcore_axis_name="s")

_sc_params = pltpu.CompilerParams()
if "needs_layout_passes" in pltpu.CompilerParams.__dataclass_fields__:
    _sc_params = dataclasses.replace(_sc_params, needs_layout_passes=False)


@functools.partial(
    pl.kernel,
    mesh=_sc_mesh,
    compiler_params=_sc_params,
    out_type=(
        jax.ShapeDtypeStruct((_B, _D), jnp.float32),
        jax.ShapeDtypeStruct((_B, _D), jnp.float32),
    ),
    scratch_types=[
        pltpu.VMEM((2, _GRP * _C, 10), jnp.int32),
        pltpu.VMEM((_BPT,), jnp.int32),
        pltpu.VMEM((2, 10 * _C), jnp.int32),
        pltpu.VMEM((2, 10 * _C), jnp.int32),
        pltpu.VMEM((2, 10 * _C, _D), jnp.float32),
        pltpu.VMEM((2, _C, _D), jnp.float32),
        pltpu.VMEM((2, _SCH, _D), jnp.float32),
        pltpu.SemaphoreType.DMA,
        pltpu.SemaphoreType.DMA,
        pltpu.SemaphoreType.DMA,
        pltpu.SemaphoreType.DMA,
        pltpu.SemaphoreType.DMA,
        pltpu.SemaphoreType.DMA,
        pltpu.SemaphoreType.DMA,
        pltpu.SemaphoreType.DMA,
    ],
)
def _sc_gather_sum(table_hbm, nidx_hbm, sidx_hbm, pat_hbm,
                   self_hbm, nsum_hbm,
                   gidx, sidx_v, pat_v, fidx, bufs, obuf, sbuf,
                   g0, g1, o0, o1, sg, sw, i0, i1):
    cid = lax.axis_index("c")
    sid = lax.axis_index("s")
    wid = sid * _NC + cid

    pltpu.sync_copy(sidx_hbm.at[pl.ds(wid * _BPT, _BPT)], sidx_v)
    pltpu.sync_copy(pat_hbm, pat_v)

    def idx_load(g, q, sem):
        return pltpu.make_async_copy(
            nidx_hbm.at[pl.ds(wid * _BPT + g * (_GRP * _C), _GRP * _C)],
            gidx.at[q], sem)

    def flatten_idx(c, p, q):
        rbase = (c % _GRP) * _C
        for k in range(10 * _C // 16):
            sl = pl.ds(k * 16, 16)
            rv = pat_v[0, sl] + rbase
            cv = pat_v[1, sl]
            fidx[p, sl] = plsc.load_gather(gidx.at[q], [rv, cv])

    def ngather(c, p, sem):
        return pltpu.make_async_copy(
            table_hbm.at[fidx.at[p]],
            bufs.at[p], sem)

    def sgather(c, p, sem):
        return pltpu.make_async_copy(
            table_hbm.at[sidx_v.at[pl.ds(c * _SCH, _SCH)]], sbuf.at[p], sem)

    def swrite(c, p, sem):
        return pltpu.make_async_copy(
            sbuf.at[p], self_hbm.at[pl.ds(wid * _BPT + c * _SCH, _SCH)], sem)

    def copyout(c, p, sem):
        return pltpu.make_async_copy(
            obuf.at[p],
            nsum_hbm.at[pl.ds(wid * _BPT + c * _C, _C)], sem)

    def sum_rows(p):
        buf = bufs.at[p]
        ob = obuf.at[p]

        def tree(x):
            t0 = x[0] + x[1]
            t1 = x[2] + x[3]
            t2 = x[4] + x[5]
            t3 = x[6] + x[7]
            t4 = x[8] + x[9]
            return ((t0 + t1) + (t2 + t3)) + t4

        @pl.loop(0, _C)
        def _(b):
            r0 = b * 10
            for k in range(0, _D // 16, 2):
                sl0 = pl.ds(k * 16, 16)
                sl1 = pl.ds((k + 1) * 16, 16)
                xa = [buf[r0 + s, sl0] for s in range(10)]
                xb = [buf[r0 + s, sl1] for s in range(10)]
                va = tree(xa)
                vb = tree(xb)
                ob[b, sl0] = va
                ob[b, sl1] = vb

    idx_load(0, 0, i0).start()
    idx_load(0, 0, i0).wait()
    idx_load(1, 1, i1).start()
    flatten_idx(0, 0, 0)
    ngather(0, 0, g0).start()
    flatten_idx(1, 1, 0)
    ngather(1, 1, g1).start()

    @pl.loop(0, _NSTEP // 2)
    def _(i):
        for p, gs, os in ((0, g0, o0), (1, g1, o1)):
            c = i * 2 + p
            ngather(c, p, gs).wait()

            @pl.when(i > 0)
            def _():
                copyout(c - 2, p, os).wait()

            sum_rows(p)
            copyout(c, p, os).start()
            nc = c + 2

            @pl.when(nc < _NSTEP)
            def _():
                ng = nc // _GRP
                for qn, s_this, s_next in ((0, i0, i1), (1, i1, i0)):
                    @pl.when(jnp.logical_and(nc % _GRP == 0,
                                             ng % 2 == qn))
                    def _():
                        idx_load(ng, qn, s_this).wait()

                        @pl.when(ng + 1 < _NGRP)
                        def _():
                            idx_load(ng + 1, 1 - qn, s_next).start()

                for qf in (0, 1):
                    @pl.when(ng % 2 == qf)
                    def _():
                        flatten_idx(nc, p, qf)
                        ngather(nc, p, gs).start()

    copyout(_NSTEP - 2, 0, o0).wait()
    copyout(_NSTEP - 1, 1, o1).wait()

    sgather(0, 0, sg).start()
    sgather(1, 1, sw).start()

    @pl.loop(0, _NSCH // 2)
    def _(i):
        for p, gs, ws in ((0, sg, o0), (1, sw, o1)):
            c = i * 2 + p
            sgather(c, p, gs).wait()
            swrite(c, p, ws).start()
            swrite(c, p, ws).wait()
            nc = c + 2

            @pl.when(nc < _NSCH)
            def _():
                sgather(nc, p, gs).start()


def _tc_body(s_ref, n_ref, w_ref, o_ref):
    self_f = s_ref[...]
    neigh = n_ref[...] * jnp.float32(0.1)
    w = w_ref[...]
    acc = lax.dot_general(
        w[:, :_D], self_f, (((1,), (1,)), ((), ())),
        preferred_element_type=jnp.float32, precision=lax.Precision.DEFAULT)
    acc = acc + lax.dot_general(
        w[:, _D:], neigh, (((1,), (1,)), ((), ())),
        preferred_element_type=jnp.float32, precision=lax.Precision.DEFAULT)
    o_ref[...] = jnp.maximum(acc, jnp.float32(0.0))


def _tc_matmul(self_rows, nsum_rows, weight):
    return pl.pallas_call(
        _tc_body,
        grid=(_NBT,),
        in_specs=[
            pl.BlockSpec((_BT, _D), lambda i: (i, 0)),
            pl.BlockSpec((_BT, _D), lambda i: (i, 0)),
            pl.BlockSpec((_E, 2 * _D), lambda i: (0, 0)),
        ],
        out_specs=pl.BlockSpec((_E, _BT), lambda i: (0, i)),
        out_shape=jax.ShapeDtypeStruct((_E, _B), jnp.float32),
    )(self_rows, nsum_rows, weight)


def kernel(features, nodes, neigh_idx, weight):
    ar = jnp.arange(10 * _C, dtype=jnp.int32)
    pat = jnp.stack([ar // 10, ar % 10])
    self_rows, nsum_rows = _sc_gather_sum(features, neigh_idx, nodes, pat)
    return _tc_matmul(self_rows, nsum_rows, weight)


# speedup vs baseline: 1.2872x; 1.0299x over previous
import dataclasses
import functools

import jax
import jax.numpy as jnp
from jax import lax
from jax.experimental import pallas as pl
from jax.experimental.pallas import tpu as pltpu
from jax.experimental.pallas import tpu_sc as plsc

_B = 16384
_D = 256
_E = 256
_NC = 2
_NS = 16
_NW = _NC * _NS
_BPT = _B // _NW
_C = 8
_NSTEP = _BPT // _C
_SCH = 64
_NSCH = _BPT // _SCH
_GRP = 8
_NGRP = _NSTEP // _GRP

_BT = 2048
_NBT = _B // _BT

_sc_mesh = plsc.VectorSubcoreMesh(core_axis_name="c", subcore_axis_name="s")

_sc_params = pltpu.CompilerParams()
if "needs_layout_passes" in pltpu.CompilerParams.__dataclass_fields__:
    _sc_params = dataclasses.replace(_sc_params, needs_layout_passes=False)


@functools.partial(
    pl.kernel,
    mesh=_sc_mesh,
    compiler_params=_sc_params,
    out_type=(
        jax.ShapeDtypeStruct((_B, _D), jnp.float32),
        jax.ShapeDtypeStruct((_B, _D), jnp.float32),
    ),
    scratch_types=[
        pltpu.VMEM((2, _GRP * _C, 10), jnp.int32),
        pltpu.VMEM((_BPT,), jnp.int32),
        pltpu.VMEM((2, 10 * _C), jnp.int32),
        pltpu.VMEM((2, 10 * _C), jnp.int32),
        pltpu.VMEM((2, 10 * _C, _D), jnp.float32),
        pltpu.VMEM((2, _C, _D), jnp.float32),
        pltpu.VMEM((2, _SCH, _D), jnp.float32),
        pltpu.SemaphoreType.DMA,
        pltpu.SemaphoreType.DMA,
        pltpu.SemaphoreType.DMA,
        pltpu.SemaphoreType.DMA,
        pltpu.SemaphoreType.DMA,
        pltpu.SemaphoreType.DMA,
        pltpu.SemaphoreType.DMA,
        pltpu.SemaphoreType.DMA,
    ],
)
def _sc_gather_sum(table_hbm, nidx_hbm, sidx_hbm, pat_hbm,
                   self_hbm, nsum_hbm,
                   gidx, sidx_v, pat_v, fidx, bufs, obuf, sbuf,
                   g0, g1, o0, o1, sg, sw, i0, i1):
    cid = lax.axis_index("c")
    sid = lax.axis_index("s")
    wid = sid * _NC + cid

    def prologue_loads():
        pltpu.sync_copy(sidx_hbm.at[pl.ds(wid * _BPT, _BPT)], sidx_v)
        pltpu.sync_copy(pat_hbm, pat_v)

    def idx_load(g, q, sem):
        return pltpu.make_async_copy(
            nidx_hbm.at[pl.ds(wid * _BPT + g * (_GRP * _C), _GRP * _C)],
            gidx.at[q], sem)

    def flatten_idx(c, p, q):
        rbase = (c % _GRP) * _C
        for k in range(10 * _C // 16):
            sl = pl.ds(k * 16, 16)
            rv = pat_v[0, sl] + rbase
            cv = pat_v[1, sl]
            fidx[p, sl] = plsc.load_gather(gidx.at[q], [rv, cv])

    def ngather(c, p, sem):
        return pltpu.make_async_copy(
            table_hbm.at[fidx.at[p]],
            bufs.at[p], sem)

    def sgather(c, p, sem):
        return pltpu.make_async_copy(
            table_hbm.at[sidx_v.at[pl.ds(c * _SCH, _SCH)]], sbuf.at[p], sem)

    def swrite(c, p, sem):
        return pltpu.make_async_copy(
            sbuf.at[p], self_hbm.at[pl.ds(wid * _BPT + c * _SCH, _SCH)], sem)

    def copyout(c, p, sem):
        return pltpu.make_async_copy(
            obuf.at[p],
            nsum_hbm.at[pl.ds(wid * _BPT + c * _C, _C)], sem)

    def sum_rows(p):
        buf = bufs.at[p]
        ob = obuf.at[p]

        def tree(x):
            t0 = x[0] + x[1]
            t1 = x[2] + x[3]
            t2 = x[4] + x[5]
            t3 = x[6] + x[7]
            t4 = x[8] + x[9]
            return ((t0 + t1) + (t2 + t3)) + t4

        @pl.loop(0, _C)
        def _(b):
            r0 = b * 10
            accs = []
            for k in range(_D // 16):
                sl = pl.ds(k * 16, 16)
                accs.append(tree([buf[r0 + s, sl] for s in range(10)]))
            for k in range(_D // 16):
                ob[b, pl.ds(k * 16, 16)] = accs[k]

    idx_load(0, 0, i0).start()
    idx_load(1, 1, i1).start()
    prologue_loads()
    idx_load(0, 0, i0).wait()
    flatten_idx(0, 0, 0)
    ngather(0, 0, g0).start()
    flatten_idx(1, 1, 0)
    ngather(1, 1, g1).start()

    @pl.loop(0, _NSTEP // 2)
    def _(i):
        for p, gs, os in ((0, g0, o0), (1, g1, o1)):
            c = i * 2 + p
            ngather(c, p, gs).wait()

            @pl.when(i > 0)
            def _():
                copyout(c - 2, p, os).wait()

            sum_rows(p)
            copyout(c, p, os).start()
            nc = c + 2

            @pl.when(nc < _NSTEP)
            def _():
                ng = nc // _GRP
                for qn, s_this, s_next in ((0, i0, i1), (1, i1, i0)):
                    @pl.when(jnp.logical_and(nc % _GRP == 0,
                                             ng % 2 == qn))
                    def _():
                        idx_load(ng, qn, s_this).wait()

                        @pl.when(ng + 1 < _NGRP)
                        def _():
                            idx_load(ng + 1, 1 - qn, s_next).start()

                for qf in (0, 1):
                    @pl.when(ng % 2 == qf)
                    def _():
                        flatten_idx(nc, p, qf)
                        ngather(nc, p, gs).start()

    copyout(_NSTEP - 2, 0, o0).wait()
    copyout(_NSTEP - 1, 1, o1).wait()

    sgather(0, 0, sg).start()
    sgather(1, 1, sw).start()

    @pl.loop(0, _NSCH // 2)
    def _(i):
        for p, gs, ws in ((0, sg, o0), (1, sw, o1)):
            c = i * 2 + p
            sgather(c, p, gs).wait()
            swrite(c, p, ws).start()
            swrite(c, p, ws).wait()
            nc = c + 2

            @pl.when(nc < _NSCH)
            def _():
                sgather(nc, p, gs).start()


def _tc_body(s_ref, n_ref, w_ref, o_ref):
    self_f = s_ref[...]
    neigh = n_ref[...] * jnp.float32(0.1)
    w = w_ref[...]
    acc = lax.dot_general(
        w[:, :_D], self_f, (((1,), (1,)), ((), ())),
        preferred_element_type=jnp.float32, precision=lax.Precision.DEFAULT)
    acc = acc + lax.dot_general(
        w[:, _D:], neigh, (((1,), (1,)), ((), ())),
        preferred_element_type=jnp.float32, precision=lax.Precision.DEFAULT)
    o_ref[...] = jnp.maximum(acc, jnp.float32(0.0))


def _tc_matmul(self_rows, nsum_rows, weight):
    return pl.pallas_call(
        _tc_body,
        grid=(_NBT,),
        in_specs=[
            pl.BlockSpec((_BT, _D), lambda i: (i, 0)),
            pl.BlockSpec((_BT, _D), lambda i: (i, 0)),
            pl.BlockSpec((_E, 2 * _D), lambda i: (0, 0)),
        ],
        out_specs=pl.BlockSpec((_E, _BT), lambda i: (0, i)),
        out_shape=jax.ShapeDtypeStruct((_E, _B), jnp.float32),
    )(self_rows, nsum_rows, weight)


def kernel(features, nodes, neigh_idx, weight):
    ar = jnp.arange(10 * _C, dtype=jnp.int32)
    pat = jnp.stack([ar // 10, ar % 10])
    self_rows, nsum_rows = _sc_gather_sum(features, neigh_idx, nodes, pat)
    return _tc_matmul(self_rows, nsum_rows, weight)


# speedup vs baseline: 1.3020x; 1.0115x over previous
import dataclasses
import functools

import jax
import jax.numpy as jnp
from jax import lax
from jax.experimental import pallas as pl
from jax.experimental.pallas import tpu as pltpu
from jax.experimental.pallas import tpu_sc as plsc

_B = 16384
_D = 256
_E = 256
_NC = 2
_NS = 16
_NW = _NC * _NS
_BPT = _B // _NW
_C = 8
_NSTEP = _BPT // _C
_SCH = 64
_NSCH = _BPT // _SCH
_GRP = 8
_NGRP = _NSTEP // _GRP

_BT = 4096
_NBT = _B // _BT

_sc_mesh = plsc.VectorSubcoreMesh(core_axis_name="c", subcore_axis_name="s")

_sc_params = pltpu.CompilerParams()
if "needs_layout_passes" in pltpu.CompilerParams.__dataclass_fields__:
    _sc_params = dataclasses.replace(_sc_params, needs_layout_passes=False)


@functools.partial(
    pl.kernel,
    mesh=_sc_mesh,
    compiler_params=_sc_params,
    out_type=(
        jax.ShapeDtypeStruct((_B, _D), jnp.float32),
        jax.ShapeDtypeStruct((_B, _D), jnp.float32),
    ),
    scratch_types=[
        pltpu.VMEM((2, _GRP * _C, 10), jnp.int32),
        pltpu.VMEM((_BPT,), jnp.int32),
        pltpu.VMEM((2, 10 * _C), jnp.int32),
        pltpu.VMEM((2, 10 * _C), jnp.int32),
        pltpu.VMEM((2, 10 * _C, _D), jnp.float32),
        pltpu.VMEM((2, _C, _D), jnp.float32),
        pltpu.VMEM((2, _SCH, _D), jnp.float32),
        pltpu.SemaphoreType.DMA,
        pltpu.SemaphoreType.DMA,
        pltpu.SemaphoreType.DMA,
        pltpu.SemaphoreType.DMA,
        pltpu.SemaphoreType.DMA,
        pltpu.SemaphoreType.DMA,
        pltpu.SemaphoreType.DMA,
        pltpu.SemaphoreType.DMA,
    ],
)
def _sc_gather_sum(table_hbm, nidx_hbm, sidx_hbm, pat_hbm,
                   self_hbm, nsum_hbm,
                   gidx, sidx_v, pat_v, fidx, bufs, obuf, sbuf,
                   g0, g1, o0, o1, sg, sw, i0, i1):
    cid = lax.axis_index("c")
    sid = lax.axis_index("s")
    wid = sid * _NC + cid

    def prologue_loads():
        pltpu.sync_copy(sidx_hbm.at[pl.ds(wid * _BPT, _BPT)], sidx_v)
        pltpu.sync_copy(pat_hbm, pat_v)

    def idx_load(g, q, sem):
        return pltpu.make_async_copy(
            nidx_hbm.at[pl.ds(wid * _BPT + g * (_GRP * _C), _GRP * _C)],
            gidx.at[q], sem)

    def flatten_idx(c, p, q):
        rbase = (c % _GRP) * _C
        for k in range(10 * _C // 16):
            sl = pl.ds(k * 16, 16)
            rv = pat_v[0, sl] + rbase
            cv = pat_v[1, sl]
            fidx[p, sl] = plsc.load_gather(gidx.at[q], [rv, cv])

    def ngather(c, p, sem):
        return pltpu.make_async_copy(
            table_hbm.at[fidx.at[p]],
            bufs.at[p], sem)

    def sgather(c, p, sem):
        return pltpu.make_async_copy(
            table_hbm.at[sidx_v.at[pl.ds(c * _SCH, _SCH)]], sbuf.at[p], sem)

    def swrite(c, p, sem):
        return pltpu.make_async_copy(
            sbuf.at[p], self_hbm.at[pl.ds(wid * _BPT + c * _SCH, _SCH)], sem)

    def copyout(c, p, sem):
        return pltpu.make_async_copy(
            obuf.at[p],
            nsum_hbm.at[pl.ds(wid * _BPT + c * _C, _C)], sem)

    def sum_rows(p):
        buf = bufs.at[p]
        ob = obuf.at[p]

        def tree(x):
            t0 = x[0] + x[1]
            t1 = x[2] + x[3]
            t2 = x[4] + x[5]
            t3 = x[6] + x[7]
            t4 = x[8] + x[9]
            return ((t0 + t1) + (t2 + t3)) + t4

        @pl.loop(0, _C)
        def _(b):
            r0 = b * 10
            accs = []
            for k in range(_D // 16):
                sl = pl.ds(k * 16, 16)
                accs.append(tree([buf[r0 + s, sl] for s in range(10)]))
            for k in range(_D // 16):
                ob[b, pl.ds(k * 16, 16)] = accs[k]

    idx_load(0, 0, i0).start()
    idx_load(1, 1, i1).start()
    prologue_loads()
    idx_load(0, 0, i0).wait()
    flatten_idx(0, 0, 0)
    ngather(0, 0, g0).start()
    flatten_idx(1, 1, 0)
    ngather(1, 1, g1).start()

    @pl.loop(0, _NSTEP // 2)
    def _(i):
        @pl.when(i == _NSTEP // 2 - 2)
        def _():
            sgather(0, 0, sg).start()
            sgather(1, 1, sw).start()

        for p, gs, os in ((0, g0, o0), (1, g1, o1)):
            c = i * 2 + p
            ngather(c, p, gs).wait()

            @pl.when(i > 0)
            def _():
                copyout(c - 2, p, os).wait()

            sum_rows(p)
            copyout(c, p, os).start()
            nc = c + 2

            @pl.when(nc < _NSTEP)
            def _():
                ng = nc // _GRP
                for qn, s_this, s_next in ((0, i0, i1), (1, i1, i0)):
                    @pl.when(jnp.logical_and(nc % _GRP == 0,
                                             ng % 2 == qn))
                    def _():
                        idx_load(ng, qn, s_this).wait()

                        @pl.when(ng + 1 < _NGRP)
                        def _():
                            idx_load(ng + 1, 1 - qn, s_next).start()

                for qf in (0, 1):
                    @pl.when(ng % 2 == qf)
                    def _():
                        flatten_idx(nc, p, qf)
                        ngather(nc, p, gs).start()

    copyout(_NSTEP - 2, 0, o0).wait()
    copyout(_NSTEP - 1, 1, o1).wait()

    @pl.loop(0, _NSCH // 2)
    def _(i):
        for p, gs, ws in ((0, sg, o0), (1, sw, o1)):
            c = i * 2 + p
            sgather(c, p, gs).wait()
            swrite(c, p, ws).start()
            swrite(c, p, ws).wait()
            nc = c + 2

            @pl.when(nc < _NSCH)
            def _():
                sgather(nc, p, gs).start()


def _tc_body(s_ref, n_ref, w_ref, o_ref):
    self_f = s_ref[...]
    neigh = n_ref[...] * jnp.float32(0.1)
    w = w_ref[...]
    acc = lax.dot_general(
        w[:, :_D], self_f, (((1,), (1,)), ((), ())),
        preferred_element_type=jnp.float32, precision=lax.Precision.DEFAULT)
    acc = acc + lax.dot_general(
        w[:, _D:], neigh, (((1,), (1,)), ((), ())),
        preferred_element_type=jnp.float32, precision=lax.Precision.DEFAULT)
    o_ref[...] = jnp.maximum(acc, jnp.float32(0.0))


def _tc_matmul(self_rows, nsum_rows, weight):
    return pl.pallas_call(
        _tc_body,
        grid=(_NBT,),
        in_specs=[
            pl.BlockSpec((_BT, _D), lambda i: (i, 0)),
            pl.BlockSpec((_BT, _D), lambda i: (i, 0)),
            pl.BlockSpec((_E, 2 * _D), lambda i: (0, 0)),
        ],
        out_specs=pl.BlockSpec((_E, _BT), lambda i: (0, i)),
        out_shape=jax.ShapeDtypeStruct((_E, _B), jnp.float32),
    )(self_rows, nsum_rows, weight)


def kernel(features, nodes, neigh_idx, weight):
    ar = jnp.arange(10 * _C, dtype=jnp.int32)
    pat = jnp.stack([ar // 10, ar % 10])
    self_rows, nsum_rows = _sc_gather_sum(features, neigh_idx, nodes, pat)
    return _tc_matmul(self_rows, nsum_rows, weight)
